# Initial kernel scaffold; baseline (speedup 1.0000x reference)
import jax
import jax.numpy as jnp
from jax.experimental import pallas as pl


def kernel(x, system_id, W, b):
    raise NotImplementedError("write your pallas kernel here")


# speedup vs baseline: 1.5882x; 1.5882x over previous
import jax
import jax.numpy as jnp
from jax.experimental import pallas as pl
from jax.experimental.pallas import tpu as pltpu

B, T, D, E, C = 4096, 16, 1024, 8, 256
BLK = 128


def _fused_kernel(sid_ref, x_ref, w_ref, b_ref, out_ref):
    xp = jnp.sum(x_ref[...], axis=1) * (1.0 / T)
    acc = jax.lax.dot_general(
        xp, w_ref[...],
        dimension_numbers=(((1,), (1,)), ((), ())),
        preferred_element_type=jnp.float32,
    )
    sid = sid_ref[0, 0, :]
    out = jnp.zeros((BLK, C), dtype=jnp.float32)
    for e in range(E):
        mask = (sid == e).astype(jnp.float32)[:, None]
        out = out + mask * (acc[:, e * C:(e + 1) * C] + b_ref[e, :][None, :])
    out_ref[...] = out


def kernel(x, system_id, W, b):
    nblk = B // BLK
    sid3 = system_id.astype(jnp.int32).reshape(nblk, 1, BLK)
    wcat = W.reshape(E * C, D)
    grid = (nblk,)
    return pl.pallas_call(
        _fused_kernel,
        grid=grid,
        in_specs=[
            pl.BlockSpec((1, 1, BLK), lambda g: (g, 0, 0)),
            pl.BlockSpec((BLK, T, D), lambda g: (g, 0, 0)),
            pl.BlockSpec((E * C, D), lambda g: (0, 0)),
            pl.BlockSpec((E, C), lambda g: (0, 0)),
        ],
        out_specs=pl.BlockSpec((BLK, C), lambda g: (g, 0)),
        out_shape=jax.ShapeDtypeStruct((B, C), jnp.float32),
        compiler_params=pltpu.CompilerParams(
            dimension_semantics=("arbitrary",),
        ),
    )(sid3, x, wcat, b)


# speedup vs baseline: 1.9066x; 1.2005x over previous
import jax
import jax.numpy as jnp
from jax.experimental import pallas as pl
from jax.experimental.pallas import tpu as pltpu

B, T, D, E, C = 4096, 16, 1024, 8, 256
BLK = 256


def _fused_kernel(sid_ref, x_ref, w_ref, b_ref, out_ref):
    xp = jnp.sum(x_ref[...], axis=1) * (1.0 / T)
    acc = jax.lax.dot_general(
        xp, w_ref[...],
        dimension_numbers=(((1,), (1,)), ((), ())),
        preferred_element_type=jnp.float32,
    )
    sid = sid_ref[0, 0, :]
    out = jnp.zeros((BLK, C), dtype=jnp.float32)
    for e in range(E):
        mask = (sid == e).astype(jnp.float32)[:, None]
        out = out + mask * (acc[:, e * C:(e + 1) * C] + b_ref[e, :][None, :])
    out_ref[...] = out


def kernel(x, system_id, W, b):
    nblk = B // BLK
    sid3 = system_id.astype(jnp.int32).reshape(nblk, 1, BLK)
    wcat = W.reshape(E * C, D)
    grid = (nblk,)
    return pl.pallas_call(
        _fused_kernel,
        grid=grid,
        in_specs=[
            pl.BlockSpec((1, 1, BLK), lambda g: (g, 0, 0)),
            pl.BlockSpec((BLK, T, D), lambda g: (g, 0, 0)),
            pl.BlockSpec((E * C, D), lambda g: (0, 0)),
            pl.BlockSpec((E, C), lambda g: (0, 0)),
        ],
        out_specs=pl.BlockSpec((BLK, C), lambda g: (g, 0)),
        out_shape=jax.ShapeDtypeStruct((B, C), jnp.float32),
        compiler_params=pltpu.CompilerParams(
            dimension_semantics=("arbitrary",),
        ),
    )(sid3, x, wcat, b)
